# baseline (speedup 1.0000x reference)
import functools

import jax
import jax.numpy as jnp
from jax import lax
from jax.experimental import pallas as pl
from jax.experimental.pallas import tpu as pltpu
from jax.experimental.pallas import tpu_sc as plsc

N = 50000
E = 800000
IN_DIM = 1433
HIDDEN = 64
OUT_DIM = 7

D2 = 32
CHUNK = 128
GA = 2
SG = 3 * GA
GD = 3
NROWS = 6432
EPAD = NROWS * CHUNK
ROWS_PER_TILE = NROWS // 16
NT_AGG = ROWS_PER_TILE // SG
ROWS_PER_TILE_DEG = NROWS // 32
NGROUPS_DEG = ROWS_PER_TILE_DEG // GD
NP2 = 50048
ACC_ROWS = NP2
ZROWS = ACC_ROWS // 16

BLK = 2176
GRID = NP2 // BLK

_mesh = lambda: plsc.VectorSubcoreMesh(core_axis_name="c", subcore_axis_name="s")


def _copy_out(acc, out_hbm, c, s):
    pltpu.sync_copy(acc.at[pl.ds(s * ZROWS, ZROWS)],
                    out_hbm.at[pl.ds(c * NP2 + s * ZROWS, ZROWS)])


@functools.partial(
    pl.kernel,
    out_type=jax.ShapeDtypeStruct((2 * NP2, 16), jnp.float32),
    mesh=_mesh(),
    scratch_types=[
        pltpu.VMEM((GD, CHUNK), jnp.int32),
        pltpu.VMEM((CHUNK, 16), jnp.float32),
        pltpu.VMEM_SHARED((ACC_ROWS, 16), jnp.float32),
        pltpu.SemaphoreType.DMA,
    ],
    compiler_params=pltpu.CompilerParams(use_tc_tiling_on_sc=False),
)
def _deg_kernel(dst_hbm, z_hbm, out_hbm, dst_v, ones_v, acc, sem):
    c = lax.axis_index("c")
    s = lax.axis_index("s")
    tid = c * 16 + s
    one = jnp.full((16,), 1.0, dtype=jnp.float32)
    for j in range(CHUNK):
        ones_v[j] = one
    pltpu.sync_copy(z_hbm.at[pl.ds(s * ZROWS, ZROWS)],
                    acc.at[pl.ds(s * ZROWS, ZROWS)])
    plsc.subcore_barrier()

    def body(gi, carry):
        rb = tid * ROWS_PER_TILE_DEG + gi * GD
        pltpu.sync_copy(dst_hbm.at[pl.ds(rb, GD)], dst_v)
        descs = [pltpu.async_copy(ones_v, acc.at[dst_v.at[j]], sem, add=True)
                 for j in range(GD)]
        for d in descs:
            d.wait()
        return carry

    lax.fori_loop(0, NGROUPS_DEG, body, 0)
    plsc.subcore_barrier()
    _copy_out(acc, out_hbm, c, s)


@functools.partial(
    pl.kernel,
    out_type=jax.ShapeDtypeStruct((2 * NP2, D2), jnp.float32),
    mesh=_mesh(),
    scratch_types=[
        pltpu.VMEM((2 * SG, CHUNK), jnp.int32),
        pltpu.VMEM((2 * SG, CHUNK), jnp.int32),
        pltpu.VMEM((SG, CHUNK, D2), jnp.float32),
        pltpu.VMEM_SHARED((ACC_ROWS, D2), jnp.float32),
        pltpu.SemaphoreType.DMA,
        pltpu.SemaphoreType.DMA,
        pltpu.SemaphoreType.DMA,
        pltpu.SemaphoreType.DMA,
        pltpu.SemaphoreType.DMA,
    ],
    compiler_params=pltpu.CompilerParams(use_tc_tiling_on_sc=False),
)
def _agg_kernel(src_hbm, dst_hbm, g_hbm, z_hbm, out_hbm,
                srci, dsti, rows_v, acc, sem_g0, sem_g1, sem_g2, sem_s,
                sem_i):
    c = lax.axis_index("c")
    s = lax.axis_index("s")
    sem_g = [sem_g0, sem_g1, sem_g2]
    pltpu.sync_copy(z_hbm.at[pl.ds(s * ZROWS, ZROWS)],
                    acc.at[pl.ds(s * ZROWS, ZROWS)])
    plsc.subcore_barrier()
    base = s * ROWS_PER_TILE

    pltpu.sync_copy(src_hbm.at[c].at[pl.ds(base, SG)], srci.at[pl.ds(0, SG)])
    pltpu.sync_copy(dst_hbm.at[pl.ds(base, SG)], dsti.at[pl.ds(0, SG)])
    for j in range(2 * GA):
        pltpu.async_copy(g_hbm.at[srci.at[j]], rows_v.at[j], sem_g[j // GA])

    def body(t, carry):
        p = lax.rem(t, 2)
        ib = p * SG
        ibn = SG - ib
        not_last = t < NT_AGG - 1

        @pl.when(not_last)
        def _():
            rbn = base + (t + 1) * SG
            pltpu.async_copy(src_hbm.at[c].at[pl.ds(rbn, SG)],
                             srci.at[pl.ds(ibn, SG)], sem_i)
            pltpu.async_copy(dst_hbm.at[pl.ds(rbn, SG)],
                             dsti.at[pl.ds(ibn, SG)], sem_i)

        for k in range(3):
            kn = (k + 2) % 3
            for j in range(GA):
                pltpu.make_async_copy(g_hbm.at[srci.at[ib + GA * k + j]],
                                      rows_v.at[GA * k + j], sem_g[k]).wait()
            for j in range(GA):
                pltpu.async_copy(rows_v.at[GA * k + j],
                                 acc.at[dsti.at[ib + GA * k + j]], sem_s,
                                 add=True)

            if k == 0:
                for j in range(GA):
                    pltpu.make_async_copy(rows_v.at[GA * kn + j],
                                          acc.at[dsti.at[ib]], sem_s).wait()
                for j in range(GA):
                    pltpu.async_copy(g_hbm.at[srci.at[ib + 2 * GA + j]],
                                     rows_v.at[GA * kn + j], sem_g[kn])
            else:
                @pl.when(not_last)
                def _(k=k, kn=kn, ib=ib, ibn=ibn):
                    if k == 1:
                        pltpu.make_async_copy(
                            src_hbm.at[c].at[pl.ds(base, SG)],
                            srci.at[pl.ds(ibn, SG)], sem_i).wait()
                        pltpu.make_async_copy(
                            dst_hbm.at[pl.ds(base, SG)],
                            dsti.at[pl.ds(ibn, SG)], sem_i).wait()
                    for j in range(GA):
                        pltpu.make_async_copy(rows_v.at[GA * kn + j],
                                              acc.at[dsti.at[ib]],
                                              sem_s).wait()
                    for j in range(GA):
                        pltpu.async_copy(
                            g_hbm.at[srci.at[ibn + GA * (k - 1) + j]],
                            rows_v.at[GA * kn + j], sem_g[kn])
        return carry

    lax.fori_loop(0, NT_AGG, body, 0)
    for j in range(2 * GA):
        pltpu.make_async_copy(rows_v.at[j], acc.at[dsti.at[0]], sem_s).wait()
    plsc.subcore_barrier()
    _copy_out(acc, out_hbm, c, s)


def _dis(deg_ref):
    p = deg_ref[0, :, 0:1] + deg_ref[1, :, 0:1] + 1.0
    return lax.rsqrt(p)


def _mm(a, b):
    return jnp.dot(a, b, preferred_element_type=jnp.float32,
                   precision=lax.Precision.DEFAULT)


def _tc1_body(xt_ref, deg_ref, w_ref, o_ref):
    dis = _dis(deg_ref)
    h = lax.dot_general(xt_ref[...], w_ref[...], (((0,), (0,)), ((), ())),
                        precision=lax.Precision.DEFAULT,
                        preferred_element_type=jnp.float32)
    g = h * dis
    o_ref[0] = g[:, :D2]
    o_ref[1] = g[:, D2:]


def _tc2_body(agg_ref, g_ref, deg_ref, w_ref, b_ref, o_ref):
    dis = _dis(deg_ref)
    agg = jnp.concatenate([agg_ref[0], agg_ref[1]], axis=1)
    g = jnp.concatenate([g_ref[0], g_ref[1]], axis=1)
    z = jax.nn.relu(dis * (agg + g) + b_ref[...])
    g2 = _mm(z, w_ref[...]) * dis
    o_ref[0] = g2[:, :D2]
    o_ref[1] = g2[:, D2:]


def _tc3_body(agg_ref, g_ref, deg_ref, w_ref, b_ref, bo_ref, o_ref):
    dis = _dis(deg_ref)
    agg = jnp.concatenate([agg_ref[0], agg_ref[1]], axis=1)
    g = jnp.concatenate([g_ref[0], g_ref[1]], axis=1)
    z = jax.nn.relu(dis * (agg + g) + b_ref[...])
    logits = _mm(z, w_ref[...]) + bo_ref[...]
    m = jnp.max(logits, axis=1, keepdims=True)
    p = jnp.exp(logits - m)
    p = p / jnp.sum(p, axis=1, keepdims=True)
    o_ref[...] = p.T


_half_spec = pl.BlockSpec((2, BLK, D2), lambda i: (0, i, 0))
_deg_spec = pl.BlockSpec((2, BLK, 16), lambda i: (0, i, 0))


def _tc1(xt, deg3, W1):
    return pl.pallas_call(
        _tc1_body,
        grid=(GRID,),
        in_specs=[
            pl.BlockSpec((IN_DIM, BLK), lambda i: (0, i)),
            _deg_spec,
            pl.BlockSpec((IN_DIM, HIDDEN), lambda i: (0, 0)),
        ],
        out_specs=_half_spec,
        out_shape=jax.ShapeDtypeStruct((2, NP2, D2), jnp.float32),
    )(xt, deg3, W1)


def _tc2(agg3, g3, deg3, W2, b1):
    return pl.pallas_call(
        _tc2_body,
        grid=(GRID,),
        in_specs=[
            _half_spec,
            _half_spec,
            _deg_spec,
            pl.BlockSpec((HIDDEN, HIDDEN), lambda i: (0, 0)),
            pl.BlockSpec((1, HIDDEN), lambda i: (0, 0)),
        ],
        out_specs=_half_spec,
        out_shape=jax.ShapeDtypeStruct((2, NP2, D2), jnp.float32),
    )(agg3, g3, deg3, W2, b1)


def _tc3(agg3, g3, deg3, Wout, b2, bout):
    return pl.pallas_call(
        _tc3_body,
        grid=(GRID,),
        in_specs=[
            _half_spec,
            _half_spec,
            _deg_spec,
            pl.BlockSpec((HIDDEN, OUT_DIM), lambda i: (0, 0)),
            pl.BlockSpec((1, HIDDEN), lambda i: (0, 0)),
            pl.BlockSpec((1, OUT_DIM), lambda i: (0, 0)),
        ],
        out_specs=pl.BlockSpec((OUT_DIM, BLK), lambda i: (0, i)),
        out_shape=jax.ShapeDtypeStruct((OUT_DIM, NP2), jnp.float32),
    )(agg3, g3, deg3, Wout, b2, bout)


def _run_impl(x, edge_index, W1, b1, W2, b2, Wout, bout):
    src = edge_index[0]
    dst = edge_index[1]
    pad = EPAD - E
    srcp = jnp.concatenate([src, jnp.zeros((pad,), jnp.int32)])
    dstp = jnp.concatenate([dst, jnp.full((pad,), N, jnp.int32)])
    src_both = jnp.stack([srcp, srcp + NP2]).reshape(2, NROWS, CHUNK)
    dst_rows = dstp.reshape(NROWS, CHUNK)
    z32 = jnp.zeros((ACC_ROWS, D2), jnp.float32)
    z16 = jnp.zeros((ACC_ROWS, 16), jnp.float32)

    deg3 = _deg_kernel(dst_rows, z16).reshape(2, NP2, 16)
    g1 = _tc1(jnp.swapaxes(x, 0, 1), deg3, W1)
    agg1 = _agg_kernel(src_both, dst_rows, g1.reshape(2 * NP2, D2),
                       z32).reshape(2, NP2, D2)
    g2 = _tc2(agg1, g1, deg3, W2, b1.reshape(1, HIDDEN))
    agg2 = _agg_kernel(src_both, dst_rows, g2.reshape(2 * NP2, D2),
                       z32).reshape(2, NP2, D2)
    outT = _tc3(agg2, g2, deg3, Wout, b2.reshape(1, HIDDEN),
                bout.reshape(1, OUT_DIM))
    return jnp.swapaxes(outT[:, :N], 0, 1)


_run = jax.jit(_run_impl)


def kernel(x, edge_index, W1, b1, W2, b2, Wout, bout):
    return _run(x, edge_index, W1, b1, W2, b2, Wout, bout)


# speedup vs baseline: 1.0208x; 1.0208x over previous
import functools

import jax
import jax.numpy as jnp
from jax import lax
from jax.experimental import pallas as pl
from jax.experimental.pallas import tpu as pltpu
from jax.experimental.pallas import tpu_sc as plsc

N = 50000
E = 800000
IN_DIM = 1433
HIDDEN = 64
OUT_DIM = 7

D2 = 32
CHUNK = 128
GA = 3
SG = 2 * GA
GD = 8
NROWS = 6432
EPAD = NROWS * CHUNK
ROWS_PER_TILE = NROWS // 16
NT_AGG = ROWS_PER_TILE // SG
ROWS_PER_TILE_DEG = 6400 // 32
NGROUPS_DEG = ROWS_PER_TILE_DEG // GD
NP2 = 50048
ACC_ROWS = NP2
ZROWS = ACC_ROWS // 16

BLK = 2176
GRID = NP2 // BLK

_mesh = lambda: plsc.VectorSubcoreMesh(core_axis_name="c", subcore_axis_name="s")


def _copy_out(acc, out_hbm, c, s):
    pltpu.sync_copy(acc.at[pl.ds(s * ZROWS, ZROWS)],
                    out_hbm.at[pl.ds(c * NP2 + s * ZROWS, ZROWS)])


@functools.partial(
    pl.kernel,
    out_type=jax.ShapeDtypeStruct((2 * NP2, 16), jnp.float32),
    mesh=_mesh(),
    scratch_types=[
        pltpu.VMEM((GD, CHUNK), jnp.int32),
        pltpu.VMEM((CHUNK, 16), jnp.float32),
        pltpu.VMEM_SHARED((ACC_ROWS, 16), jnp.float32),
        pltpu.SemaphoreType.DMA,
    ],
    compiler_params=pltpu.CompilerParams(use_tc_tiling_on_sc=False),
)
def _deg_kernel(dst_hbm, z_hbm, out_hbm, dst_v, ones_v, acc, sem):
    c = lax.axis_index("c")
    s = lax.axis_index("s")
    tid = c * 16 + s
    one = jnp.full((16,), 1.0, dtype=jnp.float32)
    for j in range(CHUNK):
        ones_v[j] = one
    pltpu.sync_copy(z_hbm.at[pl.ds(s * ZROWS, ZROWS)],
                    acc.at[pl.ds(s * ZROWS, ZROWS)])
    plsc.subcore_barrier()

    def body(gi, carry):
        rb = tid * ROWS_PER_TILE_DEG + gi * GD
        pltpu.sync_copy(dst_hbm.at[pl.ds(rb, GD)], dst_v)
        descs = [pltpu.async_copy(ones_v, acc.at[dst_v.at[j]], sem, add=True)
                 for j in range(GD)]
        for d in descs:
            d.wait()
        return carry

    lax.fori_loop(0, NGROUPS_DEG, body, 0)
    plsc.subcore_barrier()
    _copy_out(acc, out_hbm, c, s)


@functools.partial(
    pl.kernel,
    out_type=jax.ShapeDtypeStruct((2 * NP2, D2), jnp.float32),
    mesh=_mesh(),
    scratch_types=[
        pltpu.VMEM((2 * SG, CHUNK), jnp.int32),
        pltpu.VMEM((2 * SG, CHUNK), jnp.int32),
        pltpu.VMEM((SG, CHUNK, D2), jnp.float32),
        pltpu.VMEM_SHARED((ACC_ROWS, D2), jnp.float32),
        pltpu.SemaphoreType.DMA,
        pltpu.SemaphoreType.DMA,
        pltpu.SemaphoreType.DMA,
        pltpu.SemaphoreType.DMA,
    ],
    compiler_params=pltpu.CompilerParams(use_tc_tiling_on_sc=False),
)
def _agg_kernel(src_hbm, dst_hbm, g_hbm, z_hbm, out_hbm,
                srci, dsti, rows_v, acc, sem_g0, sem_g1, sem_s, sem_i):
    c = lax.axis_index("c")
    s = lax.axis_index("s")
    pltpu.sync_copy(z_hbm.at[pl.ds(s * ZROWS, ZROWS)],
                    acc.at[pl.ds(s * ZROWS, ZROWS)])
    plsc.subcore_barrier()
    base = s * ROWS_PER_TILE

    pltpu.sync_copy(src_hbm.at[c].at[pl.ds(base, SG)], srci.at[pl.ds(0, SG)])
    pltpu.sync_copy(dst_hbm.at[pl.ds(base, SG)], dsti.at[pl.ds(0, SG)])
    for j in range(GA):
        pltpu.async_copy(g_hbm.at[srci.at[j]], rows_v.at[j], sem_g0)

    def body(t, carry):
        p = lax.rem(t, 2)
        ib = p * SG
        ibn = SG - ib
        not_last = t < NT_AGG - 1

        @pl.when(not_last)
        def _():
            rbn = base + (t + 1) * SG
            pltpu.async_copy(src_hbm.at[c].at[pl.ds(rbn, SG)],
                             srci.at[pl.ds(ibn, SG)], sem_i)
            pltpu.async_copy(dst_hbm.at[pl.ds(rbn, SG)],
                             dsti.at[pl.ds(ibn, SG)], sem_i)

        for j in range(GA):
            pltpu.async_copy(g_hbm.at[srci.at[ib + GA + j]],
                             rows_v.at[GA + j], sem_g1)
        for j in range(GA):
            pltpu.make_async_copy(g_hbm.at[srci.at[ib + j]], rows_v.at[j],
                                  sem_g0).wait()
        sa = [pltpu.async_copy(rows_v.at[j], acc.at[dsti.at[ib + j]], sem_s,
                               add=True) for j in range(GA)]
        for d in sa:
            d.wait()

        @pl.when(not_last)
        def _():
            pltpu.make_async_copy(src_hbm.at[c].at[pl.ds(base, SG)],
                                  srci.at[pl.ds(ibn, SG)], sem_i).wait()
            pltpu.make_async_copy(dst_hbm.at[pl.ds(base, SG)],
                                  dsti.at[pl.ds(ibn, SG)], sem_i).wait()
            for j in range(GA):
                pltpu.async_copy(g_hbm.at[srci.at[ibn + j]], rows_v.at[j],
                                 sem_g0)

        for j in range(GA):
            pltpu.make_async_copy(g_hbm.at[srci.at[ib + GA + j]],
                                  rows_v.at[GA + j], sem_g1).wait()
        sb = [pltpu.async_copy(rows_v.at[GA + j], acc.at[dsti.at[ib + GA + j]],
                               sem_s, add=True) for j in range(GA)]
        for d in sb:
            d.wait()
        return carry

    lax.fori_loop(0, NT_AGG, body, 0)
    plsc.subcore_barrier()
    _copy_out(acc, out_hbm, c, s)


def _dis(deg_ref):
    p = deg_ref[0, :, 0:1] + deg_ref[1, :, 0:1] + 1.0
    return lax.rsqrt(p)


def _mm(a, b):
    return jnp.dot(a, b, preferred_element_type=jnp.float32,
                   precision=lax.Precision.DEFAULT)


def _tc1_body(xt_ref, deg_ref, w_ref, o_ref):
    dis = _dis(deg_ref)
    h = lax.dot_general(xt_ref[...], w_ref[...], (((0,), (0,)), ((), ())),
                        precision=lax.Precision.DEFAULT,
                        preferred_element_type=jnp.float32)
    g = h * dis
    o_ref[0] = g[:, :D2]
    o_ref[1] = g[:, D2:]


def _tc2_body(agg_ref, g_ref, deg_ref, w_ref, b_ref, o_ref):
    dis = _dis(deg_ref)
    agg = jnp.concatenate([agg_ref[0], agg_ref[1]], axis=1)
    g = jnp.concatenate([g_ref[0], g_ref[1]], axis=1)
    z = jax.nn.relu(dis * (agg + g) + b_ref[...])
    g2 = _mm(z, w_ref[...]) * dis
    o_ref[0] = g2[:, :D2]
    o_ref[1] = g2[:, D2:]


def _tc3_body(agg_ref, g_ref, deg_ref, w_ref, b_ref, bo_ref, o_ref):
    dis = _dis(deg_ref)
    agg = jnp.concatenate([agg_ref[0], agg_ref[1]], axis=1)
    g = jnp.concatenate([g_ref[0], g_ref[1]], axis=1)
    z = jax.nn.relu(dis * (agg + g) + b_ref[...])
    logits = _mm(z, w_ref[...]) + bo_ref[...]
    m = jnp.max(logits, axis=1, keepdims=True)
    p = jnp.exp(logits - m)
    p = p / jnp.sum(p, axis=1, keepdims=True)
    o_ref[...] = p.T


_half_spec = pl.BlockSpec((2, BLK, D2), lambda i: (0, i, 0))
_deg_spec = pl.BlockSpec((2, BLK, 16), lambda i: (0, i, 0))


def _tc1(xt, deg3, W1):
    return pl.pallas_call(
        _tc1_body,
        grid=(GRID,),
        in_specs=[
            pl.BlockSpec((IN_DIM, BLK), lambda i: (0, i)),
            _deg_spec,
            pl.BlockSpec((IN_DIM, HIDDEN), lambda i: (0, 0)),
        ],
        out_specs=_half_spec,
        out_shape=jax.ShapeDtypeStruct((2, NP2, D2), jnp.float32),
    )(xt, deg3, W1)


def _tc2(agg3, g3, deg3, W2, b1):
    return pl.pallas_call(
        _tc2_body,
        grid=(GRID,),
        in_specs=[
            _half_spec,
            _half_spec,
            _deg_spec,
            pl.BlockSpec((HIDDEN, HIDDEN), lambda i: (0, 0)),
            pl.BlockSpec((1, HIDDEN), lambda i: (0, 0)),
        ],
        out_specs=_half_spec,
        out_shape=jax.ShapeDtypeStruct((2, NP2, D2), jnp.float32),
    )(agg3, g3, deg3, W2, b1)


def _tc3(agg3, g3, deg3, Wout, b2, bout):
    return pl.pallas_call(
        _tc3_body,
        grid=(GRID,),
        in_specs=[
            _half_spec,
            _half_spec,
            _deg_spec,
            pl.BlockSpec((HIDDEN, OUT_DIM), lambda i: (0, 0)),
            pl.BlockSpec((1, HIDDEN), lambda i: (0, 0)),
            pl.BlockSpec((1, OUT_DIM), lambda i: (0, 0)),
        ],
        out_specs=pl.BlockSpec((OUT_DIM, BLK), lambda i: (0, i)),
        out_shape=jax.ShapeDtypeStruct((OUT_DIM, NP2), jnp.float32),
    )(agg3, g3, deg3, Wout, b2, bout)


def _run_impl(x, edge_index, W1, b1, W2, b2, Wout, bout):
    src = edge_index[0]
    dst = edge_index[1]
    pad = EPAD - E
    srcp = jnp.concatenate([src, jnp.zeros((pad,), jnp.int32)])
    dstp = jnp.concatenate([dst, jnp.full((pad,), N, jnp.int32)])
    src_both = jnp.stack([srcp, srcp + NP2]).reshape(2, NROWS, CHUNK)
    dst_rows = dstp.reshape(NROWS, CHUNK)
    z32 = jnp.zeros((ACC_ROWS, D2), jnp.float32)
    z16 = jnp.zeros((ACC_ROWS, 16), jnp.float32)

    deg3 = _deg_kernel(dst_rows, z16).reshape(2, NP2, 16)
    g1 = _tc1(jnp.swapaxes(x, 0, 1), deg3, W1)
    agg1 = _agg_kernel(src_both, dst_rows, g1.reshape(2 * NP2, D2),
                       z32).reshape(2, NP2, D2)
    g2 = _tc2(agg1, g1, deg3, W2, b1.reshape(1, HIDDEN))
    agg2 = _agg_kernel(src_both, dst_rows, g2.reshape(2 * NP2, D2),
                       z32).reshape(2, NP2, D2)
    outT = _tc3(agg2, g2, deg3, Wout, b2.reshape(1, HIDDEN),
                bout.reshape(1, OUT_DIM))
    return jnp.swapaxes(outT[:, :N], 0, 1)


_run = jax.jit(_run_impl)


def kernel(x, edge_index, W1, b1, W2, b2, Wout, bout):
    return _run(x, edge_index, W1, b1, W2, b2, Wout, bout)
